# baseline (speedup 1.0000x reference)
import functools
import jax
import jax.numpy as jnp
from jax import lax
from jax.experimental import pallas as pl
from jax.experimental.pallas import tpu as pltpu
from jax.experimental.pallas import tpu_sc as plsc

NSEG = 104
EB = 2560

SC_NC = 2
SC_NS = 16
SC_NW = SC_NC * SC_NS
GCH = 80


NBUF = 5


PKW = 64


def _sc_gather_body(xe_hbm, h_hbm, t_hbm, outh_hbm, outt_hbm,
                    idxh_a, idxt_a, rowsh_v, rowst_v, *sems):
    sg = sems[:NBUF]
    sw = sems[NBUF:]
    wid = lax.axis_index("s") * SC_NC + lax.axis_index("c")
    per_w = h_hbm.shape[0] // SC_NW
    base = wid * per_w
    pltpu.sync_copy(h_hbm.at[pl.ds(base, per_w)], idxh_a)
    pltpu.sync_copy(t_hbm.at[pl.ds(base, per_w)], idxt_a)

    def grp(g, carry):
        gbase = g * (NBUF * GCH)
        copies = []
        for b in range(NBUF):
            @pl.when(g > 0)
            def _(b=b):
                pltpu.make_async_copy(
                    outh_hbm.at[pl.ds(0, GCH)],
                    rowsh_v.at[pl.ds(b * GCH, GCH)], sw[b]).wait()
                pltpu.make_async_copy(
                    outt_hbm.at[pl.ds(0, GCH)],
                    rowst_v.at[pl.ds(b * GCH, GCH)], sw[b]).wait()

            off = gbase + b * GCH
            cph = pltpu.async_copy(
                xe_hbm.at[idxh_a.at[pl.ds(off, GCH)]],
                rowsh_v.at[pl.ds(b * GCH, GCH)], sg[b])
            cpt = pltpu.async_copy(
                xe_hbm.at[idxt_a.at[pl.ds(off, GCH)]],
                rowst_v.at[pl.ds(b * GCH, GCH)], sg[b])
            copies.append((cph, cpt))
        for b, (cph, cpt) in enumerate(copies):
            cph.wait()
            cpt.wait()
            off = gbase + b * GCH
            pltpu.async_copy(rowsh_v.at[pl.ds(b * GCH, GCH)],
                             outh_hbm.at[pl.ds(base + off, GCH)], sw[b])
            pltpu.async_copy(rowst_v.at[pl.ds(b * GCH, GCH)],
                             outt_hbm.at[pl.ds(base + off, GCH)], sw[b])
        return carry

    lax.fori_loop(0, per_w // (GCH * NBUF), grp, None)
    for b in range(NBUF):
        pltpu.make_async_copy(outh_hbm.at[pl.ds(0, GCH)],
                              rowsh_v.at[pl.ds(b * GCH, GCH)], sw[b]).wait()
        pltpu.make_async_copy(outt_hbm.at[pl.ds(0, GCH)],
                              rowst_v.at[pl.ds(b * GCH, GCH)], sw[b]).wait()


def _sc_gather(xe_bf, h, t):
    E = h.shape[0]
    E_HID = xe_bf.shape[1]
    per_w = E // SC_NW
    mesh = plsc.VectorSubcoreMesh(core_axis_name="c", subcore_axis_name="s")
    fn = functools.partial(
        pl.kernel,
        mesh=mesh,
        compiler_params=pltpu.CompilerParams(use_tc_tiling_on_sc=False),
        out_type=[
            jax.ShapeDtypeStruct((E, PKW), jnp.int32),
            jax.ShapeDtypeStruct((E, PKW), jnp.int32),
        ],
        scratch_types=[
            pltpu.VMEM((per_w,), jnp.int32),
            pltpu.VMEM((per_w,), jnp.int32),
            pltpu.VMEM((NBUF * GCH, PKW), jnp.int32),
            pltpu.VMEM((NBUF * GCH, PKW), jnp.int32),
        ] + [pltpu.SemaphoreType.DMA] * (2 * NBUF),
    )(_sc_gather_body)
    return fn(xe_bf, h, t)


def _k1_body(rel_ref, xh_ref, xt_ref, u_ref, c_ref, acc_ref):
    xh = xh_ref[...]
    xt = xt_ref[...]
    xh32 = xh.astype(jnp.float32)
    xt32 = xt.astype(jnp.float32)
    rel_row = rel_ref[0]
    u1h = u_ref[0:1, :]
    u1t = u_ref[1:2, :]
    u2h = u_ref[2:3, :]
    u2t = u_ref[3:4, :]
    e1 = jnp.sum(xh32 * u1h, axis=1, keepdims=True) + jnp.sum(xt32 * u1t, axis=1, keepdims=True) + c_ref[0, 0]
    e2 = jnp.sum(xh32 * u2h, axis=1, keepdims=True) + jnp.sum(xt32 * u2t, axis=1, keepdims=True) + c_ref[0, 1]
    z1 = jnp.exp(jnp.where(e1 > 0, e1, 0.01 * e1))
    z2 = jnp.exp(jnp.where(e2 > 0, e2, 0.01 * e2))
    eb = xh.shape[0]
    ones = jnp.ones((eb, 1), jnp.bfloat16)
    pad = jnp.zeros((eb, 125), jnp.bfloat16)
    scl = jnp.concatenate([ones, z1.astype(jnp.bfloat16), z2.astype(jnp.bfloat16), pad], axis=1)
    rhs = jnp.concatenate([scl, xh, xt,
                           (z1 * xh32).astype(jnp.bfloat16),
                           (z2 * xt32).astype(jnp.bfloat16)], axis=1)
    onehot_t = (lax.broadcasted_iota(jnp.int32, (NSEG, eb), 0) == rel_row).astype(jnp.bfloat16)
    upd = lax.dot_general(onehot_t, rhs, (((1,), (0,)), ((), ())),
                          preferred_element_type=jnp.float32)

    @pl.when(pl.program_id(0) == 0)
    def _():
        acc_ref[...] = jnp.zeros_like(acc_ref)

    acc_ref[...] += upd


def _k2_body(acc_ref, wsr_ref, bsr_ref, wh_ref, wt_ref, bsr1_ref, c_out_ref):
    acc = acc_ref[...]
    cnt = acc[:, 0:1]
    z1 = acc[:, 1:2]
    z2 = acc[:, 2:3]
    sh = acc[:, 128:256]
    st = acc[:, 256:384]
    g1 = acc[:, 384:512]
    g2 = acc[:, 512:640]
    cntc = jnp.maximum(cnt, 1.0)
    mean_h = sh / cntc
    mean_t = st / cntc
    wsr = wsr_ref[...]
    res = (lax.dot_general(mean_h, wsr[:, :128], (((1,), (1,)), ((), ())), preferred_element_type=jnp.float32)
           + lax.dot_general(mean_t, wsr[:, 128:], (((1,), (1,)), ((), ())), preferred_element_type=jnp.float32)
           + bsr_ref[...])
    gh1 = jnp.where(z1 > 0, g1 / jnp.maximum(z1, 1e-30), 0.0)
    gt2 = jnp.where(z2 > 0, g2 / jnp.maximum(z2, 1e-30), 0.0)
    xr1 = lax.dot_general(gh1, wh_ref[...], (((1,), (1,)), ((), ())), preferred_element_type=jnp.float32)
    xr2 = lax.dot_general(gt2, wt_ref[...], (((1,), (1,)), ((), ())), preferred_element_type=jnp.float32)
    mask = (cnt > 0).astype(jnp.float32)
    c_out_ref[...] = xr1 + xr2 + res * (1.0 + mask) + bsr1_ref[...]


def _k3_body(rel_ref, xh_ref, xt_ref, c_ref, a_ref, b_ref, out_ref):
    rel_row = rel_ref[0]
    eb = xh_ref.shape[0]
    onehot_t = (lax.broadcasted_iota(jnp.int32, (NSEG, eb), 0) == rel_row).astype(jnp.bfloat16)
    cbf = c_ref[...].astype(jnp.bfloat16)
    crow = lax.dot_general(onehot_t, cbf, (((0,), (0,)), ((), ())), preferred_element_type=jnp.float32)
    ta = lax.dot_general(xt_ref[...], a_ref[...], (((1,), (1,)), ((), ())), preferred_element_type=jnp.float32)
    hb = lax.dot_general(xh_ref[...], b_ref[...], (((1,), (1,)), ((), ())), preferred_element_type=jnp.float32)
    out_ref[...] = crow + ta + hb


def kernel(x_e, edge_index, rel, rel_size, W_ah1, W_ah2, W_at1, W_at2, W_ah3, W_at3, W_h, W_t, W_sr, b_sr, W_sr1, b_sr1):
    N, E_HID = x_e.shape
    E = rel.shape[0]
    h = edge_index[0]
    t = edge_index[1]
    A = W_sr1[:, :E_HID]
    B = W_sr1[:, E_HID:]
    a1 = W_ah1.reshape(-1)
    a2 = W_ah2.reshape(-1)
    t1 = W_at1.reshape(-1)
    t2 = W_at2.reshape(-1)
    u1h = W_h.T @ (a1 + a2) + B.T @ a2
    u1t = (A.T + W_t.T) @ a2
    u2h = (W_h.T + B.T) @ t1
    u2t = A.T @ t1 + W_t.T @ (t1 + t2)
    u = jnp.stack([u1h, u1t, u2h, u2t], axis=0)
    c12 = jnp.stack([b_sr1 @ a2, b_sr1 @ t1]).reshape(1, 2)

    xe_bf = x_e.astype(jnp.bfloat16)
    xe_pk = lax.bitcast_convert_type(xe_bf.reshape(N, PKW, 2), jnp.int32)
    xh_pk, xt_pk = _sc_gather(xe_pk, h, t)
    xh = lax.bitcast_convert_type(xh_pk, jnp.bfloat16).reshape(E, E_HID)
    xt = lax.bitcast_convert_type(xt_pk, jnp.bfloat16).reshape(E, E_HID)

    nblk = E // EB
    rel3 = rel.reshape(nblk, 1, EB)
    acc = pl.pallas_call(
        _k1_body,
        grid=(nblk,),
        in_specs=[
            pl.BlockSpec((1, 1, EB), lambda i: (i, 0, 0)),
            pl.BlockSpec((EB, E_HID), lambda i: (i, 0)),
            pl.BlockSpec((EB, E_HID), lambda i: (i, 0)),
            pl.BlockSpec((4, E_HID), lambda i: (0, 0)),
            pl.BlockSpec((1, 2), lambda i: (0, 0)),
        ],
        out_specs=pl.BlockSpec((NSEG, 640), lambda i: (0, 0)),
        out_shape=jax.ShapeDtypeStruct((NSEG, 640), jnp.float32),
    )(rel3, xh, xt, u, c12)

    c_tab = pl.pallas_call(
        _k2_body,
        in_specs=[pl.BlockSpec(s.shape, lambda: tuple(0 for _ in s.shape)) for s in (
            jax.ShapeDtypeStruct((NSEG, 640), jnp.float32),
            jax.ShapeDtypeStruct((E_HID, 2 * E_HID), jnp.float32),
            jax.ShapeDtypeStruct((1, E_HID), jnp.float32),
            jax.ShapeDtypeStruct((E_HID, E_HID), jnp.float32),
            jax.ShapeDtypeStruct((E_HID, E_HID), jnp.float32),
            jax.ShapeDtypeStruct((1, E_HID), jnp.float32),
        )],
        out_specs=pl.BlockSpec((NSEG, E_HID), lambda: (0, 0)),
        out_shape=jax.ShapeDtypeStruct((NSEG, E_HID), jnp.float32),
    )(acc, W_sr, b_sr.reshape(1, -1), W_h, W_t, b_sr1.reshape(1, -1))

    out = pl.pallas_call(
        _k3_body,
        grid=(nblk,),
        in_specs=[
            pl.BlockSpec((1, 1, EB), lambda i: (i, 0, 0)),
            pl.BlockSpec((EB, E_HID), lambda i: (i, 0)),
            pl.BlockSpec((EB, E_HID), lambda i: (i, 0)),
            pl.BlockSpec((NSEG, E_HID), lambda i: (0, 0)),
            pl.BlockSpec((E_HID, E_HID), lambda i: (0, 0)),
            pl.BlockSpec((E_HID, E_HID), lambda i: (0, 0)),
        ],
        out_specs=pl.BlockSpec((EB, E_HID), lambda i: (i, 0)),
        out_shape=jax.ShapeDtypeStruct((E, E_HID), jnp.float32),
    )(rel3, xh, xt, c_tab, A.astype(jnp.bfloat16), B.astype(jnp.bfloat16))
    return out


# speedup vs baseline: 3.0707x; 3.0707x over previous
import functools
import jax
import jax.numpy as jnp
from jax import lax
from jax.experimental import pallas as pl
from jax.experimental.pallas import tpu as pltpu
from jax.experimental.pallas import tpu_sc as plsc

NSEG = 104
EB = 2560

SC_NC = 2
SC_NS = 16
SC_NW = SC_NC * SC_NS
GCH = 80


NBUF = 5


PKW = 64


def _sc_gather_body(xe_hbm, h_hbm, t_hbm, outh_hbm, outt_hbm,
                    idxh_a, idxt_a, rowsh_v, rowst_v, *sems):
    sg = sems[:NBUF]
    sw = sems[NBUF:]
    wid = lax.axis_index("s") * SC_NC + lax.axis_index("c")
    per_w = h_hbm.shape[0] // SC_NW
    base = wid * per_w
    pltpu.sync_copy(h_hbm.at[pl.ds(base, per_w)], idxh_a)
    pltpu.sync_copy(t_hbm.at[pl.ds(base, per_w)], idxt_a)

    def grp(g, carry):
        gbase = g * (NBUF * GCH)
        copies = []
        for b in range(NBUF):
            @pl.when(g > 0)
            def _(b=b):
                pltpu.make_async_copy(
                    outh_hbm.at[pl.ds(0, GCH)],
                    rowsh_v.at[pl.ds(b * GCH, GCH)], sw[b]).wait()
                pltpu.make_async_copy(
                    outt_hbm.at[pl.ds(0, GCH)],
                    rowst_v.at[pl.ds(b * GCH, GCH)], sw[b]).wait()

            off = gbase + b * GCH
            cph = pltpu.async_copy(
                xe_hbm.at[idxh_a.at[pl.ds(off, GCH)]],
                rowsh_v.at[pl.ds(b * GCH, GCH)], sg[b])
            cpt = pltpu.async_copy(
                xe_hbm.at[idxt_a.at[pl.ds(off, GCH)]],
                rowst_v.at[pl.ds(b * GCH, GCH)], sg[b])
            copies.append((cph, cpt))
        for b, (cph, cpt) in enumerate(copies):
            cph.wait()
            cpt.wait()
            off = gbase + b * GCH
            pltpu.async_copy(rowsh_v.at[pl.ds(b * GCH, GCH)],
                             outh_hbm.at[pl.ds(base + off, GCH)], sw[b])
            pltpu.async_copy(rowst_v.at[pl.ds(b * GCH, GCH)],
                             outt_hbm.at[pl.ds(base + off, GCH)], sw[b])
        return carry

    lax.fori_loop(0, per_w // (GCH * NBUF), grp, None)
    for b in range(NBUF):
        pltpu.make_async_copy(outh_hbm.at[pl.ds(0, GCH)],
                              rowsh_v.at[pl.ds(b * GCH, GCH)], sw[b]).wait()
        pltpu.make_async_copy(outt_hbm.at[pl.ds(0, GCH)],
                              rowst_v.at[pl.ds(b * GCH, GCH)], sw[b]).wait()


def _sc_gather(xe_bf, h, t):
    E = h.shape[0]
    E_HID = xe_bf.shape[1]
    per_w = E // SC_NW
    mesh = plsc.VectorSubcoreMesh(core_axis_name="c", subcore_axis_name="s")
    fn = functools.partial(
        pl.kernel,
        mesh=mesh,
        out_type=[
            jax.ShapeDtypeStruct((E, E_HID), jnp.float32),
            jax.ShapeDtypeStruct((E, E_HID), jnp.float32),
        ],
        scratch_types=[
            pltpu.VMEM((per_w,), jnp.int32),
            pltpu.VMEM((per_w,), jnp.int32),
            pltpu.VMEM((NBUF * GCH, E_HID), jnp.float32),
            pltpu.VMEM((NBUF * GCH, E_HID), jnp.float32),
        ] + [pltpu.SemaphoreType.DMA] * (2 * NBUF),
    )(_sc_gather_body)
    return fn(xe_bf, h, t)


def _k1_body(rel_ref, xh_ref, xt_ref, u_ref, c_ref, acc_ref):
    xh32 = xh_ref[...]
    xt32 = xt_ref[...]
    xh = xh32.astype(jnp.bfloat16)
    xt = xt32.astype(jnp.bfloat16)
    rel_row = rel_ref[0]
    u1h = u_ref[0:1, :]
    u1t = u_ref[1:2, :]
    u2h = u_ref[2:3, :]
    u2t = u_ref[3:4, :]
    e1 = jnp.sum(xh32 * u1h, axis=1, keepdims=True) + jnp.sum(xt32 * u1t, axis=1, keepdims=True) + c_ref[0, 0]
    e2 = jnp.sum(xh32 * u2h, axis=1, keepdims=True) + jnp.sum(xt32 * u2t, axis=1, keepdims=True) + c_ref[0, 1]
    z1 = jnp.exp(jnp.where(e1 > 0, e1, 0.01 * e1))
    z2 = jnp.exp(jnp.where(e2 > 0, e2, 0.01 * e2))
    eb = xh.shape[0]
    ones = jnp.ones((eb, 1), jnp.bfloat16)
    pad = jnp.zeros((eb, 125), jnp.bfloat16)
    scl = jnp.concatenate([ones, z1.astype(jnp.bfloat16), z2.astype(jnp.bfloat16), pad], axis=1)
    rhs = jnp.concatenate([scl, xh, xt,
                           (z1 * xh32).astype(jnp.bfloat16),
                           (z2 * xt32).astype(jnp.bfloat16)], axis=1)
    onehot_t = (lax.broadcasted_iota(jnp.int32, (NSEG, eb), 0) == rel_row).astype(jnp.bfloat16)
    upd = lax.dot_general(onehot_t, rhs, (((1,), (0,)), ((), ())),
                          preferred_element_type=jnp.float32)

    @pl.when(pl.program_id(0) == 0)
    def _():
        acc_ref[...] = jnp.zeros_like(acc_ref)

    acc_ref[...] += upd


def _k2_body(acc_ref, wsr_ref, bsr_ref, wh_ref, wt_ref, bsr1_ref, c_out_ref):
    acc = acc_ref[...]
    cnt = acc[:, 0:1]
    z1 = acc[:, 1:2]
    z2 = acc[:, 2:3]
    sh = acc[:, 128:256]
    st = acc[:, 256:384]
    g1 = acc[:, 384:512]
    g2 = acc[:, 512:640]
    cntc = jnp.maximum(cnt, 1.0)
    mean_h = sh / cntc
    mean_t = st / cntc
    wsr = wsr_ref[...]
    res = (lax.dot_general(mean_h, wsr[:, :128], (((1,), (1,)), ((), ())), preferred_element_type=jnp.float32)
           + lax.dot_general(mean_t, wsr[:, 128:], (((1,), (1,)), ((), ())), preferred_element_type=jnp.float32)
           + bsr_ref[...])
    gh1 = jnp.where(z1 > 0, g1 / jnp.maximum(z1, 1e-30), 0.0)
    gt2 = jnp.where(z2 > 0, g2 / jnp.maximum(z2, 1e-30), 0.0)
    xr1 = lax.dot_general(gh1, wh_ref[...], (((1,), (1,)), ((), ())), preferred_element_type=jnp.float32)
    xr2 = lax.dot_general(gt2, wt_ref[...], (((1,), (1,)), ((), ())), preferred_element_type=jnp.float32)
    mask = (cnt > 0).astype(jnp.float32)
    c_out_ref[...] = xr1 + xr2 + res * (1.0 + mask) + bsr1_ref[...]


def _k3_body(rel_ref, xh_ref, xt_ref, c_ref, a_ref, b_ref, out_ref):
    rel_row = rel_ref[0]
    eb = xh_ref.shape[0]
    onehot_t = (lax.broadcasted_iota(jnp.int32, (NSEG, eb), 0) == rel_row).astype(jnp.bfloat16)
    cbf = c_ref[...].astype(jnp.bfloat16)
    crow = lax.dot_general(onehot_t, cbf, (((0,), (0,)), ((), ())), preferred_element_type=jnp.float32)
    ta = lax.dot_general(xt_ref[...].astype(jnp.bfloat16), a_ref[...], (((1,), (1,)), ((), ())), preferred_element_type=jnp.float32)
    hb = lax.dot_general(xh_ref[...].astype(jnp.bfloat16), b_ref[...], (((1,), (1,)), ((), ())), preferred_element_type=jnp.float32)
    out_ref[...] = crow + ta + hb


def kernel(x_e, edge_index, rel, rel_size, W_ah1, W_ah2, W_at1, W_at2, W_ah3, W_at3, W_h, W_t, W_sr, b_sr, W_sr1, b_sr1):
    N, E_HID = x_e.shape
    E = rel.shape[0]
    h = edge_index[0]
    t = edge_index[1]
    A = W_sr1[:, :E_HID]
    B = W_sr1[:, E_HID:]
    a1 = W_ah1.reshape(-1)
    a2 = W_ah2.reshape(-1)
    t1 = W_at1.reshape(-1)
    t2 = W_at2.reshape(-1)
    u1h = W_h.T @ (a1 + a2) + B.T @ a2
    u1t = (A.T + W_t.T) @ a2
    u2h = (W_h.T + B.T) @ t1
    u2t = A.T @ t1 + W_t.T @ (t1 + t2)
    u = jnp.stack([u1h, u1t, u2h, u2t], axis=0)
    c12 = jnp.stack([b_sr1 @ a2, b_sr1 @ t1]).reshape(1, 2)

    xh, xt = _sc_gather(x_e, h, t)

    nblk = E // EB
    rel3 = rel.reshape(nblk, 1, EB)
    acc = pl.pallas_call(
        _k1_body,
        grid=(nblk,),
        in_specs=[
            pl.BlockSpec((1, 1, EB), lambda i: (i, 0, 0)),
            pl.BlockSpec((EB, E_HID), lambda i: (i, 0)),
            pl.BlockSpec((EB, E_HID), lambda i: (i, 0)),
            pl.BlockSpec((4, E_HID), lambda i: (0, 0)),
            pl.BlockSpec((1, 2), lambda i: (0, 0)),
        ],
        out_specs=pl.BlockSpec((NSEG, 640), lambda i: (0, 0)),
        out_shape=jax.ShapeDtypeStruct((NSEG, 640), jnp.float32),
    )(rel3, xh, xt, u, c12)

    c_tab = pl.pallas_call(
        _k2_body,
        in_specs=[pl.BlockSpec(s.shape, lambda: tuple(0 for _ in s.shape)) for s in (
            jax.ShapeDtypeStruct((NSEG, 640), jnp.float32),
            jax.ShapeDtypeStruct((E_HID, 2 * E_HID), jnp.float32),
            jax.ShapeDtypeStruct((1, E_HID), jnp.float32),
            jax.ShapeDtypeStruct((E_HID, E_HID), jnp.float32),
            jax.ShapeDtypeStruct((E_HID, E_HID), jnp.float32),
            jax.ShapeDtypeStruct((1, E_HID), jnp.float32),
        )],
        out_specs=pl.BlockSpec((NSEG, E_HID), lambda: (0, 0)),
        out_shape=jax.ShapeDtypeStruct((NSEG, E_HID), jnp.float32),
    )(acc, W_sr, b_sr.reshape(1, -1), W_h, W_t, b_sr1.reshape(1, -1))

    out = pl.pallas_call(
        _k3_body,
        grid=(nblk,),
        in_specs=[
            pl.BlockSpec((1, 1, EB), lambda i: (i, 0, 0)),
            pl.BlockSpec((EB, E_HID), lambda i: (i, 0)),
            pl.BlockSpec((EB, E_HID), lambda i: (i, 0)),
            pl.BlockSpec((NSEG, E_HID), lambda i: (0, 0)),
            pl.BlockSpec((E_HID, E_HID), lambda i: (0, 0)),
            pl.BlockSpec((E_HID, E_HID), lambda i: (0, 0)),
        ],
        out_specs=pl.BlockSpec((EB, E_HID), lambda i: (i, 0)),
        out_shape=jax.ShapeDtypeStruct((E, E_HID), jnp.float32),
    )(rel3, xh, xt, c_tab, A.astype(jnp.bfloat16), B.astype(jnp.bfloat16))
    return out


# speedup vs baseline: 3.3007x; 1.0749x over previous
import functools
import jax
import jax.numpy as jnp
from jax import lax
from jax.experimental import pallas as pl
from jax.experimental.pallas import tpu as pltpu
from jax.experimental.pallas import tpu_sc as plsc

NSEG = 104
EB = 2560

SC_NC = 2
SC_NS = 16
SC_NW = SC_NC * SC_NS
GCH = 80


NBUF = 5


PKW = 64


def _sc_gather_body(xe_hbm, h_hbm, t_hbm, outh_hbm, outt_hbm,
                    idxh_a, idxt_a, rowsh_v, rowst_v, *sems):
    sg = sems[:NBUF]
    sw = sems[NBUF:]
    wid = lax.axis_index("s") * SC_NC + lax.axis_index("c")
    per_w = h_hbm.shape[0] // SC_NW
    base = wid * per_w
    pltpu.sync_copy(h_hbm.at[pl.ds(base, per_w)], idxh_a)
    pltpu.sync_copy(t_hbm.at[pl.ds(base, per_w)], idxt_a)

    def grp(g, carry):
        gbase = g * (NBUF * GCH)
        copies = []
        for b in range(NBUF):
            @pl.when(g > 0)
            def _(b=b):
                pltpu.make_async_copy(
                    outh_hbm.at[pl.ds(0, GCH)],
                    rowsh_v.at[pl.ds(b * GCH, GCH)], sw[b]).wait()
                pltpu.make_async_copy(
                    outt_hbm.at[pl.ds(0, GCH)],
                    rowst_v.at[pl.ds(b * GCH, GCH)], sw[b]).wait()

            off = gbase + b * GCH
            cph = pltpu.async_copy(
                xe_hbm.at[idxh_a.at[pl.ds(off, GCH)]],
                rowsh_v.at[pl.ds(b * GCH, GCH)], sg[b])
            cpt = pltpu.async_copy(
                xe_hbm.at[idxt_a.at[pl.ds(off, GCH)]],
                rowst_v.at[pl.ds(b * GCH, GCH)], sg[b])
            copies.append((cph, cpt))
        for b, (cph, cpt) in enumerate(copies):
            cph.wait()
            cpt.wait()
            off = gbase + b * GCH
            pltpu.async_copy(rowsh_v.at[pl.ds(b * GCH, GCH)],
                             outh_hbm.at[pl.ds(base + off, GCH)], sw[b])
            pltpu.async_copy(rowst_v.at[pl.ds(b * GCH, GCH)],
                             outt_hbm.at[pl.ds(base + off, GCH)], sw[b])
        return carry

    lax.fori_loop(0, per_w // (GCH * NBUF), grp, None)
    for b in range(NBUF):
        pltpu.make_async_copy(outh_hbm.at[pl.ds(0, GCH)],
                              rowsh_v.at[pl.ds(b * GCH, GCH)], sw[b]).wait()
        pltpu.make_async_copy(outt_hbm.at[pl.ds(0, GCH)],
                              rowst_v.at[pl.ds(b * GCH, GCH)], sw[b]).wait()


def _sc_gather(xe_bf, h, t):
    E = h.shape[0]
    E_HID = xe_bf.shape[1]
    per_w = E // SC_NW
    mesh = plsc.VectorSubcoreMesh(core_axis_name="c", subcore_axis_name="s")
    fn = functools.partial(
        pl.kernel,
        mesh=mesh,
        out_type=[
            jax.ShapeDtypeStruct((E, E_HID), jnp.float32),
            jax.ShapeDtypeStruct((E, E_HID), jnp.float32),
        ],
        scratch_types=[
            pltpu.VMEM((per_w,), jnp.int32),
            pltpu.VMEM((per_w,), jnp.int32),
            pltpu.VMEM((NBUF * GCH, E_HID), jnp.float32),
            pltpu.VMEM((NBUF * GCH, E_HID), jnp.float32),
        ] + [pltpu.SemaphoreType.DMA] * (2 * NBUF),
    )(_sc_gather_body)
    return fn(xe_bf, h, t)


def _k1_body(rel_ref, xh_ref, xt_ref, u_ref, c_ref, acc_ref):
    xh = xh_ref[...].astype(jnp.bfloat16)
    xt = xt_ref[...].astype(jnp.bfloat16)
    eb = xh.shape[0]
    rel_row = rel_ref[0]
    eh = lax.dot_general(u_ref[0:8], xh, (((1,), (1,)), ((), ())),
                         preferred_element_type=jnp.float32)
    et = lax.dot_general(u_ref[8:16], xt, (((1,), (1,)), ((), ())),
                         preferred_element_type=jnp.float32)
    er = eh + et
    e1 = er[0:1, :] + c_ref[0, 0]
    e2 = er[1:2, :] + c_ref[0, 1]
    z1r = jnp.exp(jnp.where(e1 > 0, e1, 0.01 * e1)).astype(jnp.bfloat16)
    z2r = jnp.exp(jnp.where(e2 > 0, e2, 0.01 * e2)).astype(jnp.bfloat16)
    ohb = (lax.broadcasted_iota(jnp.int32, (NSEG, eb), 0) == rel_row).astype(jnp.bfloat16)
    lhs = jnp.concatenate([ohb, ohb * z1r, ohb * z2r], axis=0)
    ones = jnp.ones((eb, 1), jnp.bfloat16)
    pad = jnp.zeros((eb, 127), jnp.bfloat16)
    rhs = jnp.concatenate([ones, pad, xh, xt], axis=1)
    upd = lax.dot_general(lhs, rhs, (((1,), (0,)), ((), ())),
                          preferred_element_type=jnp.float32)

    @pl.when(pl.program_id(0) == 0)
    def _():
        acc_ref[...] = jnp.zeros_like(acc_ref)

    acc_ref[...] += upd


def _k2_body(acc_ref, wsr_ref, bsr_ref, wh_ref, wt_ref, bsr1_ref, c_out_ref):
    acc = acc_ref[...]
    cnt = acc[0:NSEG, 0:1]
    z1 = acc[NSEG:2 * NSEG, 0:1]
    z2 = acc[2 * NSEG:3 * NSEG, 0:1]
    sh = acc[0:NSEG, 128:256]
    st = acc[0:NSEG, 256:384]
    g1 = acc[NSEG:2 * NSEG, 128:256]
    g2 = acc[2 * NSEG:3 * NSEG, 256:384]
    cntc = jnp.maximum(cnt, 1.0)
    mean_h = sh / cntc
    mean_t = st / cntc
    wsr = wsr_ref[...]
    res = (lax.dot_general(mean_h, wsr[:, :128], (((1,), (1,)), ((), ())), preferred_element_type=jnp.float32)
           + lax.dot_general(mean_t, wsr[:, 128:], (((1,), (1,)), ((), ())), preferred_element_type=jnp.float32)
           + bsr_ref[...])
    gh1 = jnp.where(z1 > 0, g1 / jnp.maximum(z1, 1e-30), 0.0)
    gt2 = jnp.where(z2 > 0, g2 / jnp.maximum(z2, 1e-30), 0.0)
    xr1 = lax.dot_general(gh1, wh_ref[...], (((1,), (1,)), ((), ())), preferred_element_type=jnp.float32)
    xr2 = lax.dot_general(gt2, wt_ref[...], (((1,), (1,)), ((), ())), preferred_element_type=jnp.float32)
    mask = (cnt > 0).astype(jnp.float32)
    c_out_ref[...] = xr1 + xr2 + res * (1.0 + mask) + bsr1_ref[...]


def _k3_body(rel_ref, xh_ref, xt_ref, c_ref, a_ref, b_ref, out_ref):
    rel_row = rel_ref[0]
    eb = xh_ref.shape[0]
    onehot_t = (lax.broadcasted_iota(jnp.int32, (NSEG, eb), 0) == rel_row).astype(jnp.bfloat16)
    cbf = c_ref[...].astype(jnp.bfloat16)
    crow = lax.dot_general(onehot_t, cbf, (((0,), (0,)), ((), ())), preferred_element_type=jnp.float32)
    ta = lax.dot_general(xt_ref[...].astype(jnp.bfloat16), a_ref[...], (((1,), (1,)), ((), ())), preferred_element_type=jnp.float32)
    hb = lax.dot_general(xh_ref[...].astype(jnp.bfloat16), b_ref[...], (((1,), (1,)), ((), ())), preferred_element_type=jnp.float32)
    out_ref[...] = crow + ta + hb


def kernel(x_e, edge_index, rel, rel_size, W_ah1, W_ah2, W_at1, W_at2, W_ah3, W_at3, W_h, W_t, W_sr, b_sr, W_sr1, b_sr1):
    N, E_HID = x_e.shape
    E = rel.shape[0]
    h = edge_index[0]
    t = edge_index[1]
    A = W_sr1[:, :E_HID]
    B = W_sr1[:, E_HID:]
    a1 = W_ah1.reshape(-1)
    a2 = W_ah2.reshape(-1)
    t1 = W_at1.reshape(-1)
    t2 = W_at2.reshape(-1)
    u1h = W_h.T @ (a1 + a2) + B.T @ a2
    u1t = (A.T + W_t.T) @ a2
    u2h = (W_h.T + B.T) @ t1
    u2t = A.T @ t1 + W_t.T @ (t1 + t2)
    uh = jnp.pad(jnp.stack([u1h, u2h], axis=0), ((0, 6), (0, 0)))
    ut = jnp.pad(jnp.stack([u1t, u2t], axis=0), ((0, 6), (0, 0)))
    u = jnp.concatenate([uh, ut], axis=0).astype(jnp.bfloat16)
    c12 = jnp.stack([b_sr1 @ a2, b_sr1 @ t1]).reshape(1, 2)

    xh, xt = _sc_gather(x_e, h, t)

    nblk = E // EB
    rel3 = rel.reshape(nblk, 1, EB)
    acc = pl.pallas_call(
        _k1_body,
        grid=(nblk,),
        in_specs=[
            pl.BlockSpec((1, 1, EB), lambda i: (i, 0, 0)),
            pl.BlockSpec((EB, E_HID), lambda i: (i, 0)),
            pl.BlockSpec((EB, E_HID), lambda i: (i, 0)),
            pl.BlockSpec((16, E_HID), lambda i: (0, 0)),
            pl.BlockSpec((1, 2), lambda i: (0, 0)),
        ],
        out_specs=pl.BlockSpec((3 * NSEG, 384), lambda i: (0, 0)),
        out_shape=jax.ShapeDtypeStruct((3 * NSEG, 384), jnp.float32),
    )(rel3, xh, xt, u, c12)

    c_tab = pl.pallas_call(
        _k2_body,
        in_specs=[pl.BlockSpec(s.shape, lambda: tuple(0 for _ in s.shape)) for s in (
            jax.ShapeDtypeStruct((3 * NSEG, 384), jnp.float32),
            jax.ShapeDtypeStruct((E_HID, 2 * E_HID), jnp.float32),
            jax.ShapeDtypeStruct((1, E_HID), jnp.float32),
            jax.ShapeDtypeStruct((E_HID, E_HID), jnp.float32),
            jax.ShapeDtypeStruct((E_HID, E_HID), jnp.float32),
            jax.ShapeDtypeStruct((1, E_HID), jnp.float32),
        )],
        out_specs=pl.BlockSpec((NSEG, E_HID), lambda: (0, 0)),
        out_shape=jax.ShapeDtypeStruct((NSEG, E_HID), jnp.float32),
    )(acc, W_sr, b_sr.reshape(1, -1), W_h, W_t, b_sr1.reshape(1, -1))

    out = pl.pallas_call(
        _k3_body,
        grid=(nblk,),
        in_specs=[
            pl.BlockSpec((1, 1, EB), lambda i: (i, 0, 0)),
            pl.BlockSpec((EB, E_HID), lambda i: (i, 0)),
            pl.BlockSpec((EB, E_HID), lambda i: (i, 0)),
            pl.BlockSpec((NSEG, E_HID), lambda i: (0, 0)),
            pl.BlockSpec((E_HID, E_HID), lambda i: (0, 0)),
            pl.BlockSpec((E_HID, E_HID), lambda i: (0, 0)),
        ],
        out_specs=pl.BlockSpec((EB, E_HID), lambda i: (i, 0)),
        out_shape=jax.ShapeDtypeStruct((E, E_HID), jnp.float32),
    )(rel3, xh, xt, c_tab, A.astype(jnp.bfloat16), B.astype(jnp.bfloat16))
    return out


# speedup vs baseline: 3.3073x; 1.0020x over previous
import functools
import jax
import jax.numpy as jnp
from jax import lax
from jax.experimental import pallas as pl
from jax.experimental.pallas import tpu as pltpu
from jax.experimental.pallas import tpu_sc as plsc

NSEG = 104
EB = 2560

SC_NC = 2
SC_NS = 16
SC_NW = SC_NC * SC_NS
GCH = 80
NBUF = 5


def _sc_gather_body(xe_hbm, h_hbm, t_hbm, outh_hbm, outt_hbm,
                    idxh_a, idxt_a, rowsh_v, rowst_v, *sems):
    sg = sems[:NBUF]
    sw = sems[NBUF:]
    wid = lax.axis_index("s") * SC_NC + lax.axis_index("c")
    per_w = h_hbm.shape[0] // SC_NW
    base = wid * per_w
    pltpu.sync_copy(h_hbm.at[pl.ds(base, per_w)], idxh_a)
    pltpu.sync_copy(t_hbm.at[pl.ds(base, per_w)], idxt_a)

    def grp(g, carry):
        gbase = g * (NBUF * GCH)
        copies = []
        for b in range(NBUF):
            @pl.when(g > 0)
            def _(b=b):
                pltpu.make_async_copy(
                    outh_hbm.at[pl.ds(0, GCH)],
                    rowsh_v.at[pl.ds(b * GCH, GCH)], sw[b]).wait()
                pltpu.make_async_copy(
                    outt_hbm.at[pl.ds(0, GCH)],
                    rowst_v.at[pl.ds(b * GCH, GCH)], sw[b]).wait()

            off = gbase + b * GCH
            cph = pltpu.async_copy(
                xe_hbm.at[idxh_a.at[pl.ds(off, GCH)]],
                rowsh_v.at[pl.ds(b * GCH, GCH)], sg[b])
            cpt = pltpu.async_copy(
                xe_hbm.at[idxt_a.at[pl.ds(off, GCH)]],
                rowst_v.at[pl.ds(b * GCH, GCH)], sg[b])
            copies.append((cph, cpt))
        for b, (cph, cpt) in enumerate(copies):
            cph.wait()
            cpt.wait()
            off = gbase + b * GCH
            pltpu.async_copy(rowsh_v.at[pl.ds(b * GCH, GCH)],
                             outh_hbm.at[pl.ds(base + off, GCH)], sw[b])
            pltpu.async_copy(rowst_v.at[pl.ds(b * GCH, GCH)],
                             outt_hbm.at[pl.ds(base + off, GCH)], sw[b])
        return carry

    lax.fori_loop(0, per_w // (GCH * NBUF), grp, None)
    for b in range(NBUF):
        pltpu.make_async_copy(outh_hbm.at[pl.ds(0, GCH)],
                              rowsh_v.at[pl.ds(b * GCH, GCH)], sw[b]).wait()
        pltpu.make_async_copy(outt_hbm.at[pl.ds(0, GCH)],
                              rowst_v.at[pl.ds(b * GCH, GCH)], sw[b]).wait()


def _sc_gather(x_e, h, t):
    E = h.shape[0]
    E_HID = x_e.shape[1]
    per_w = E // SC_NW
    mesh = plsc.VectorSubcoreMesh(core_axis_name="c", subcore_axis_name="s")
    fn = functools.partial(
        pl.kernel,
        mesh=mesh,
        out_type=[
            jax.ShapeDtypeStruct((E, E_HID), jnp.float32),
            jax.ShapeDtypeStruct((E, E_HID), jnp.float32),
        ],
        scratch_types=[
            pltpu.VMEM((per_w,), jnp.int32),
            pltpu.VMEM((per_w,), jnp.int32),
            pltpu.VMEM((NBUF * GCH, E_HID), jnp.float32),
            pltpu.VMEM((NBUF * GCH, E_HID), jnp.float32),
        ] + [pltpu.SemaphoreType.DMA] * (2 * NBUF),
    )(_sc_gather_body)
    return fn(x_e, h, t)


def _k1_body(rel_ref, xh_ref, xt_ref, u_ref, c_ref, acc_ref):
    xh = xh_ref[...].astype(jnp.bfloat16)
    xt = xt_ref[...].astype(jnp.bfloat16)
    eb = xh.shape[0]
    rel_row = rel_ref[0]
    eh = lax.dot_general(u_ref[0:8], xh, (((1,), (1,)), ((), ())),
                         preferred_element_type=jnp.float32)
    et = lax.dot_general(u_ref[8:16], xt, (((1,), (1,)), ((), ())),
                         preferred_element_type=jnp.float32)
    er = eh + et
    e1 = er[0:1, :] + c_ref[0, 0]
    e2 = er[1:2, :] + c_ref[0, 1]
    z1r = jnp.exp(jnp.where(e1 > 0, e1, 0.01 * e1)).astype(jnp.bfloat16)
    z2r = jnp.exp(jnp.where(e2 > 0, e2, 0.01 * e2)).astype(jnp.bfloat16)
    ohb = (lax.broadcasted_iota(jnp.int32, (NSEG, eb), 0) == rel_row).astype(jnp.bfloat16)
    lhs = jnp.concatenate([ohb, ohb * z1r, ohb * z2r], axis=0)
    ones = jnp.ones((eb, 1), jnp.bfloat16)
    pad = jnp.zeros((eb, 127), jnp.bfloat16)
    rhs = jnp.concatenate([ones, pad, xh, xt], axis=1)
    upd = lax.dot_general(lhs, rhs, (((1,), (0,)), ((), ())),
                          preferred_element_type=jnp.float32)

    @pl.when(pl.program_id(0) == 0)
    def _():
        acc_ref[...] = jnp.zeros_like(acc_ref)

    acc_ref[...] += upd


def _k2_body(acc_ref, wsr_ref, bsr_ref, wh_ref, wt_ref, bsr1_ref, c_out_ref):
    acc = acc_ref[...]
    cnt = acc[0:NSEG, 0:1]
    z1 = acc[NSEG:2 * NSEG, 0:1]
    z2 = acc[2 * NSEG:3 * NSEG, 0:1]
    sh = acc[0:NSEG, 128:256]
    st = acc[0:NSEG, 256:384]
    g1 = acc[NSEG:2 * NSEG, 128:256]
    g2 = acc[2 * NSEG:3 * NSEG, 256:384]
    cntc = jnp.maximum(cnt, 1.0)
    mean_h = sh / cntc
    mean_t = st / cntc
    wsr = wsr_ref[...]
    res = (lax.dot_general(mean_h, wsr[:, :128], (((1,), (1,)), ((), ())), preferred_element_type=jnp.float32)
           + lax.dot_general(mean_t, wsr[:, 128:], (((1,), (1,)), ((), ())), preferred_element_type=jnp.float32)
           + bsr_ref[...])
    gh1 = jnp.where(z1 > 0, g1 / jnp.maximum(z1, 1e-30), 0.0)
    gt2 = jnp.where(z2 > 0, g2 / jnp.maximum(z2, 1e-30), 0.0)
    xr1 = lax.dot_general(gh1, wh_ref[...], (((1,), (1,)), ((), ())), preferred_element_type=jnp.float32)
    xr2 = lax.dot_general(gt2, wt_ref[...], (((1,), (1,)), ((), ())), preferred_element_type=jnp.float32)
    mask = (cnt > 0).astype(jnp.float32)
    c_out_ref[...] = xr1 + xr2 + res * (1.0 + mask) + bsr1_ref[...]


def _k3_body(rel_ref, xh_ref, xt_ref, c_ref, a_ref, b_ref, out_ref):
    rel_row = rel_ref[0]
    eb = xh_ref.shape[0]
    onehot_t = (lax.broadcasted_iota(jnp.int32, (NSEG, eb), 0) == rel_row).astype(jnp.bfloat16)
    cbf = c_ref[...].astype(jnp.bfloat16)
    crow = lax.dot_general(onehot_t, cbf, (((0,), (0,)), ((), ())), preferred_element_type=jnp.float32)
    ta = lax.dot_general(xt_ref[...].astype(jnp.bfloat16), a_ref[...], (((1,), (1,)), ((), ())), preferred_element_type=jnp.float32)
    hb = lax.dot_general(xh_ref[...].astype(jnp.bfloat16), b_ref[...], (((1,), (1,)), ((), ())), preferred_element_type=jnp.float32)
    out_ref[...] = crow + ta + hb


def kernel(x_e, edge_index, rel, rel_size, W_ah1, W_ah2, W_at1, W_at2, W_ah3, W_at3, W_h, W_t, W_sr, b_sr, W_sr1, b_sr1):
    N, E_HID = x_e.shape
    E = rel.shape[0]
    h = edge_index[0]
    t = edge_index[1]
    A = W_sr1[:, :E_HID]
    B = W_sr1[:, E_HID:]
    a1 = W_ah1.reshape(-1)
    a2 = W_ah2.reshape(-1)
    t1 = W_at1.reshape(-1)
    t2 = W_at2.reshape(-1)
    u1h = W_h.T @ (a1 + a2) + B.T @ a2
    u1t = (A.T + W_t.T) @ a2
    u2h = (W_h.T + B.T) @ t1
    u2t = A.T @ t1 + W_t.T @ (t1 + t2)
    uh = jnp.pad(jnp.stack([u1h, u2h], axis=0), ((0, 6), (0, 0)))
    ut = jnp.pad(jnp.stack([u1t, u2t], axis=0), ((0, 6), (0, 0)))
    u = jnp.concatenate([uh, ut], axis=0).astype(jnp.bfloat16)
    c12 = jnp.stack([b_sr1 @ a2, b_sr1 @ t1]).reshape(1, 2)

    xh, xt = _sc_gather(x_e, h, t)

    nblk = E // EB
    rel3 = rel.reshape(nblk, 1, EB)
    acc = pl.pallas_call(
        _k1_body,
        grid=(nblk,),
        in_specs=[
            pl.BlockSpec((1, 1, EB), lambda i: (i, 0, 0)),
            pl.BlockSpec((EB, E_HID), lambda i: (i, 0)),
            pl.BlockSpec((EB, E_HID), lambda i: (i, 0)),
            pl.BlockSpec((16, E_HID), lambda i: (0, 0)),
            pl.BlockSpec((1, 2), lambda i: (0, 0)),
        ],
        out_specs=pl.BlockSpec((3 * NSEG, 384), lambda i: (0, 0)),
        out_shape=jax.ShapeDtypeStruct((3 * NSEG, 384), jnp.float32),
    )(rel3, xh, xt, u, c12)

    c_tab = pl.pallas_call(
        _k2_body,
        in_specs=[pl.BlockSpec(s.shape, lambda: tuple(0 for _ in s.shape)) for s in (
            jax.ShapeDtypeStruct((3 * NSEG, 384), jnp.float32),
            jax.ShapeDtypeStruct((E_HID, 2 * E_HID), jnp.float32),
            jax.ShapeDtypeStruct((1, E_HID), jnp.float32),
            jax.ShapeDtypeStruct((E_HID, E_HID), jnp.float32),
            jax.ShapeDtypeStruct((E_HID, E_HID), jnp.float32),
            jax.ShapeDtypeStruct((1, E_HID), jnp.float32),
        )],
        out_specs=pl.BlockSpec((NSEG, E_HID), lambda: (0, 0)),
        out_shape=jax.ShapeDtypeStruct((NSEG, E_HID), jnp.float32),
    )(acc, W_sr, b_sr.reshape(1, -1), W_h, W_t, b_sr1.reshape(1, -1))

    out = pl.pallas_call(
        _k3_body,
        grid=(nblk,),
        in_specs=[
            pl.BlockSpec((1, 1, EB), lambda i: (i, 0, 0)),
            pl.BlockSpec((EB, E_HID), lambda i: (i, 0)),
            pl.BlockSpec((EB, E_HID), lambda i: (i, 0)),
            pl.BlockSpec((NSEG, E_HID), lambda i: (0, 0)),
            pl.BlockSpec((E_HID, E_HID), lambda i: (0, 0)),
            pl.BlockSpec((E_HID, E_HID), lambda i: (0, 0)),
        ],
        out_specs=pl.BlockSpec((EB, E_HID), lambda i: (i, 0)),
        out_shape=jax.ShapeDtypeStruct((E, E_HID), jnp.float32),
    )(rel3, xh, xt, c_tab, A.astype(jnp.bfloat16), B.astype(jnp.bfloat16))
    return out
